# Initial kernel scaffold; baseline (speedup 1.0000x reference)
import jax
import jax.numpy as jnp
from jax.experimental import pallas as pl


def kernel(x_cat, tables, W1, b1, W2, b2, W3, b3):
    raise NotImplementedError("write your pallas kernel here")


# speedup vs baseline: 12.1966x; 12.1966x over previous
import functools

import jax
import jax.numpy as jnp
from jax import lax
from jax.experimental import pallas as pl
from jax.experimental.pallas import tpu as pltpu
from jax.experimental.pallas import tpu_sc as plsc

_F = 26
_V = 100000
_D = 32
_P = 128
_B = 16384
_R = _F * _D

_NC = 2
_NS = 16
_NW = _NC * _NS
_RPW = _R // _NW
_CHUNK = 8192
_NCHUNK = _B // _CHUNK


def _make_gather():
    mesh = plsc.VectorSubcoreMesh(core_axis_name="c", subcore_axis_name="s")

    @functools.partial(
        pl.kernel,
        mesh=mesh,
        out_type=jax.ShapeDtypeStruct((_R, _B), jnp.float32),
        scratch_types=[
            pltpu.VMEM((_V,), jnp.float32),
            pltpu.VMEM((_CHUNK,), jnp.int32),
            pltpu.VMEM((_CHUNK,), jnp.float32),
        ],
        compiler_params=pltpu.CompilerParams(
            use_tc_tiling_on_sc=False, needs_layout_passes=False),
    )
    def gather_k(idx_hbm, table_hbm, out_hbm, row_v, idx_v, out_v):
        wid = lax.axis_index("s") * _NC + lax.axis_index("c")

        def per_field(f, carry):
            r = f * _D + wid
            pltpu.sync_copy(table_hbm.at[r], row_v)

            def per_chunk(c, carry2):
                pltpu.sync_copy(idx_hbm.at[f, pl.ds(c * _CHUNK, _CHUNK)],
                                idx_v)

                def per_block(j, carry3):
                    base = j * 128
                    for u in range(8):
                        o = base + u * 16
                        idx16 = idx_v[pl.ds(o, 16)]
                        out_v[pl.ds(o, 16)] = plsc.load_gather(row_v, [idx16])
                    return carry3

                lax.fori_loop(0, _CHUNK // 128, per_block, 0)
                pltpu.sync_copy(out_v,
                                out_hbm.at[r, pl.ds(c * _CHUNK, _CHUNK)])
                return carry2

            lax.fori_loop(0, _NCHUNK, per_chunk, 0)
            return carry

        lax.fori_loop(0, _RPW, per_field, 0)

    return gather_k


_gather = _make_gather()

_BLK = 2048


def _gelu_exact(x):
    return 0.5 * x * (1.0 + lax.erf(x * 0.7071067811865476))


def _mlp_body(ht_ref, w1t_ref, b1_ref, w2t_ref, b2_ref, w3_ref, b3_ref, o_ref):
    ht = ht_ref[...]
    z = jnp.dot(w1t_ref[...], ht, preferred_element_type=jnp.float32)
    h1 = _gelu_exact(z + b1_ref[...])
    z = jnp.dot(w2t_ref[...], h1, preferred_element_type=jnp.float32)
    h2 = _gelu_exact(z + b2_ref[...])
    z = lax.dot_general(h2, w3_ref[...], (((0,), (0,)), ((), ())),
                        preferred_element_type=jnp.float32)
    o_ref[...] = z + b3_ref[...]


_mlp = pl.pallas_call(
    _mlp_body,
    grid=(_B // _BLK,),
    in_specs=[
        pl.BlockSpec((_R, _BLK), lambda i: (0, i)),
        pl.BlockSpec((_P, _R), lambda i: (0, 0)),
        pl.BlockSpec((_P, 1), lambda i: (0, 0)),
        pl.BlockSpec((_P, _P), lambda i: (0, 0)),
        pl.BlockSpec((_P, 1), lambda i: (0, 0)),
        pl.BlockSpec((_P, _P), lambda i: (0, 0)),
        pl.BlockSpec((1, _P), lambda i: (0, 0)),
    ],
    out_specs=pl.BlockSpec((_BLK, _P), lambda i: (i, 0)),
    out_shape=jax.ShapeDtypeStruct((_B, _P), jnp.float32),
)


def kernel(x_cat, tables, W1, b1, W2, b2, W3, b3):
    idx_fb = x_cat.T
    table_rows = tables.transpose(0, 2, 1).reshape(_R, _V)
    ht = _gather(idx_fb, table_rows)
    return _mlp(ht, W1.T, b1.reshape(_P, 1), W2.T, b2.reshape(_P, 1),
                W3, b3.reshape(1, _P))


# speedup vs baseline: 29.8920x; 2.4509x over previous
import functools

import jax
import jax.numpy as jnp
from jax import lax
from jax.experimental import pallas as pl
from jax.experimental.pallas import tpu as pltpu
from jax.experimental.pallas import tpu_sc as plsc

_F = 26
_V = 100000
_D = 32
_P = 128
_B = 16384
_R = _F * _D

_NC = 2
_NS = 16
_NW = _NC * _NS
_RPW = _R // _NW
_CHUNK = 8192
_NCHUNK = _B // _CHUNK


def _make_gather():
    mesh = plsc.VectorSubcoreMesh(core_axis_name="c", subcore_axis_name="s")

    @functools.partial(
        pl.kernel,
        mesh=mesh,
        out_type=jax.ShapeDtypeStruct((_R, _B), jnp.float32),
        scratch_types=[
            pltpu.VMEM((_V,), jnp.float32),
            pltpu.VMEM((_CHUNK,), jnp.int32),
            pltpu.VMEM((_CHUNK,), jnp.float32),
        ],
        compiler_params=pltpu.CompilerParams(
            use_tc_tiling_on_sc=True, needs_layout_passes=False),
    )
    def gather_k(idx_hbm, table_hbm, out_hbm, row_v, idx_v, out_v):
        wid = lax.axis_index("s") * _NC + lax.axis_index("c")

        def per_field(f, carry):
            r = f * _D + wid
            pltpu.sync_copy(table_hbm.at[r], row_v)

            def per_chunk(c, carry2):
                pltpu.sync_copy(idx_hbm.at[f, pl.ds(c * _CHUNK, _CHUNK)],
                                idx_v)

                def per_block(j, carry3):
                    base = j * 128
                    for u in range(8):
                        o = base + u * 16
                        idx16 = idx_v[pl.ds(o, 16)]
                        out_v[pl.ds(o, 16)] = plsc.load_gather(row_v, [idx16])
                    return carry3

                lax.fori_loop(0, _CHUNK // 128, per_block, 0)
                pltpu.sync_copy(out_v,
                                out_hbm.at[r, pl.ds(c * _CHUNK, _CHUNK)])
                return carry2

            lax.fori_loop(0, _NCHUNK, per_chunk, 0)
            return carry

        lax.fori_loop(0, _RPW, per_field, 0)

    return gather_k


_gather = _make_gather()

_BLK = 2048


def _gelu_exact(x):
    return 0.5 * x * (1.0 + lax.erf(x * 0.7071067811865476))


def _mlp_body(ht_ref, w1t_ref, b1_ref, w2t_ref, b2_ref, w3_ref, b3_ref, o_ref):
    ht = ht_ref[...]
    z = jnp.dot(w1t_ref[...], ht, preferred_element_type=jnp.float32)
    h1 = _gelu_exact(z + b1_ref[...])
    z = jnp.dot(w2t_ref[...], h1, preferred_element_type=jnp.float32)
    h2 = _gelu_exact(z + b2_ref[...])
    z = lax.dot_general(h2, w3_ref[...], (((0,), (0,)), ((), ())),
                        preferred_element_type=jnp.float32)
    o_ref[...] = z + b3_ref[...]


_mlp = pl.pallas_call(
    _mlp_body,
    grid=(_B // _BLK,),
    in_specs=[
        pl.BlockSpec((_R, _BLK), lambda i: (0, i)),
        pl.BlockSpec((_P, _R), lambda i: (0, 0)),
        pl.BlockSpec((_P, 1), lambda i: (0, 0)),
        pl.BlockSpec((_P, _P), lambda i: (0, 0)),
        pl.BlockSpec((_P, 1), lambda i: (0, 0)),
        pl.BlockSpec((_P, _P), lambda i: (0, 0)),
        pl.BlockSpec((1, _P), lambda i: (0, 0)),
    ],
    out_specs=pl.BlockSpec((_BLK, _P), lambda i: (i, 0)),
    out_shape=jax.ShapeDtypeStruct((_B, _P), jnp.float32),
)


def kernel(x_cat, tables, W1, b1, W2, b2, W3, b3):
    idx_fb = x_cat.T
    table_rows = tables.transpose(0, 2, 1).reshape(_R, _V)
    ht = _gather(idx_fb, table_rows)
    return _mlp(ht, W1.T, b1.reshape(_P, 1), W2.T, b2.reshape(_P, 1),
                W3, b3.reshape(1, _P))
